# baseline (speedup 1.0000x reference)
import functools

import jax
import jax.numpy as jnp
from jax import lax
from jax.experimental import pallas as pl
from jax.experimental.pallas import tpu as pltpu
from jax.experimental.pallas import tpu_sc as plsc

V = 64
D = 16
LANES = 16


def _make_sc_histogram(B, L):
    info = plsc.get_sparse_core_info()
    NC, NS = info.num_cores, info.num_subcores
    NW = NC * NS
    half_rows_w = (B // 2) // NW
    chunkb = min(half_rows_w, 256)
    n_chunks = half_rows_w // chunkb
    subb = 64
    n_groups = subb // LANES
    xtra = (L, chunkb) if n_chunks > 1 else (8, 8)

    mesh = plsc.VectorSubcoreMesh(core_axis_name="c", subcore_axis_name="s")

    @functools.partial(
        pl.kernel,
        mesh=mesh,
        compiler_params=pltpu.CompilerParams(needs_layout_passes=False),
        out_type=jax.ShapeDtypeStruct(((B // 2) * 2 * V,), jnp.float32),
        scratch_types=[
            pltpu.VMEM((L, chunkb), jnp.int32),
            pltpu.VMEM(xtra, jnp.int32),
            pltpu.VMEM((L, chunkb), jnp.int32),
            pltpu.VMEM(xtra, jnp.int32),
            pltpu.VMEM((subb * 2 * V,), jnp.float32),
            pltpu.VMEM((subb * 2 * V,), jnp.float32),
            pltpu.SemaphoreType.DMA,
            pltpu.SemaphoreType.DMA,
            pltpu.SemaphoreType.DMA,
            pltpu.SemaphoreType.DMA,
        ],
    )
    def hist(xt_hbm, out_hbm, xa0, xa1, xb0, xb1, cbuf0, cbuf1,
             in0, in1, out0, out1):
        wid = lax.axis_index("s") * NC + lax.axis_index("c")
        base2 = wid * half_rows_w
        zeros16 = jnp.zeros((LANES,), jnp.float32)
        ones16 = jnp.full((LANES,), 1.0, jnp.float32)
        row_step = lax.iota(jnp.int32, LANES) * (2 * V)
        xabufs = (xa0, xa1) if n_chunks > 1 else (xa0, xa0)
        xbbufs = (xb0, xb1) if n_chunks > 1 else (xb0, xb0)
        cbufs = (cbuf0, cbuf1)
        in_sems = (in0, in1)
        out_sems = (out0, out1)

        def start_in(c):
            col0 = base2 + c * chunkb
            sem = in_sems[c % 2]
            return [
                pltpu.async_copy(
                    xt_hbm.at[:, pl.ds(col0, chunkb)], xabufs[c % 2], sem),
                pltpu.async_copy(
                    xt_hbm.at[:, pl.ds(col0 + B // 2, chunkb)],
                    xbbufs[c % 2], sem),
            ]

        def start_out(s, sl):
            row0 = base2 + s * subb
            return pltpu.async_copy(
                cbufs[sl],
                out_hbm.at[pl.ds(row0 * 2 * V, subb * 2 * V)],
                out_sems[sl])

        pending_in = [start_in(0)]
        pending_out = [None, None]
        n_per_chunk = chunkb // subb
        n_subs = n_chunks * n_per_chunk
        zvecs = subb * 2 * V // LANES
        n_iters = L // 4
        zper = zvecs // n_iters
        ztail = zvecs - zper * n_iters

        @plsc.parallel_loop(0, zvecs, 1, unroll=8)
        def zero_body(r):
            cbuf0[pl.ds(r * LANES, LANES)] = zeros16
            cbuf1[pl.ds(r * LANES, LANES)] = zeros16

        for s in range(n_subs):
            c, sub = divmod(s, n_per_chunk)
            sl = s % 2
            if sub == 0:
                for h in pending_in[c]:
                    h.wait()
                if c + 1 < n_chunks:
                    pending_in.append(start_in(c + 1))
            zb = None
            if s >= 1 and s + 1 < n_subs:
                osl = sl ^ 1
                if pending_out[osl] is not None:
                    pending_out[osl].wait()
                    pending_out[osl] = None
                zb = cbufs[osl]
            xa = xabufs[c % 2]
            xb = xbbufs[c % 2]
            cb = cbufs[sl]

            def l_body(li, carry):
                work = []
                for u in range(4):
                    l = li * 4 + u
                    for p, xp in ((0, xa), (1, xb)):
                        for g in range(n_groups):
                            win = cb.at[pl.ds(
                                g * LANES * 2 * V + p * V,
                                (LANES - 1) * 2 * V + V)]
                            tok = xp[l, pl.ds(sub * subb + g * LANES,
                                              LANES)]
                            work.append((win, tok))
                idxs = [row_step + tok for _, tok in work]
                if zb is not None:
                    for z in range(zper):
                        zb[pl.ds((li * zper + z) * LANES, LANES)] = zeros16
                for (win, _), idx in zip(work, idxs):
                    plsc.addupdate_scatter(win, [idx], ones16)
                return carry

            lax.fori_loop(0, n_iters, l_body, 0)
            if zb is not None:
                for z in range(ztail):
                    zb[pl.ds((zper * n_iters + z) * LANES, LANES)] = zeros16
            pending_out[sl] = start_out(s, sl)
        for p in pending_out:
            if p is not None:
                p.wait()

    return hist


def _tc_head_body(inv_l, counts_ref, emb_ref, w_ref, b_ref, out_ref):
    m2 = lax.dot_general(emb_ref[...], w_ref[...],
                         (((1,), (1,)), ((), ())),
                         preferred_element_type=jnp.float32) * inv_l
    h = pl.program_id(0)
    m2s = jnp.concatenate([m2, m2], axis=0)
    rid = lax.broadcasted_iota(jnp.int32, (2 * V, V), 0)
    m2h = jnp.where((rid // V) == h, m2s, 0.0)
    out_ref[...] = lax.dot_general(m2h, counts_ref[...],
                                   (((0,), (1,)), ((), ())),
                                   preferred_element_type=jnp.float32) + b_ref[...]


def _tc_head(counts2, emb, W, bcol, L):
    Bh = counts2.shape[0]
    tb = min(Bh, 8192)
    nb = Bh // tb
    return pl.pallas_call(
        functools.partial(_tc_head_body, 1.0 / L),
        grid=(2, nb),
        in_specs=[
            pl.BlockSpec((tb, 2 * V), lambda h, j: (j, 0)),
            pl.BlockSpec((V, D), lambda h, j: (0, 0)),
            pl.BlockSpec((V, D), lambda h, j: (0, 0)),
            pl.BlockSpec((V, 1), lambda h, j: (0, 0)),
        ],
        out_specs=pl.BlockSpec((V, tb), lambda h, j: (0, h * nb + j)),
        out_shape=jax.ShapeDtypeStruct((V, 2 * Bh), jnp.float32),
    )(counts2, emb, W, bcol)


def kernel(x, emb, W, b):
    B, L = x.shape
    x = x.astype(jnp.int32)
    counts2 = _make_sc_histogram(B, L)(x.T).reshape(B // 2, 2 * V)
    out_t = _tc_head(counts2, emb, W, b.reshape(V, 1), L)
    return out_t.T


# speedup vs baseline: 1.0503x; 1.0503x over previous
import functools

import jax
import jax.numpy as jnp
from jax import lax
from jax.experimental import pallas as pl
from jax.experimental.pallas import tpu as pltpu
from jax.experimental.pallas import tpu_sc as plsc

V = 64
D = 16
LANES = 16


def _make_sc_histogram(B, L):
    info = plsc.get_sparse_core_info()
    NC, NS = info.num_cores, info.num_subcores
    NW = NC * NS
    half_rows_w = (B // 2) // NW
    chunkb = min(half_rows_w, 128)
    n_chunks = half_rows_w // chunkb
    subb = chunkb // 2
    n_groups = subb // LANES

    mesh = plsc.VectorSubcoreMesh(core_axis_name="c", subcore_axis_name="s")

    @functools.partial(
        pl.kernel,
        mesh=mesh,
        compiler_params=pltpu.CompilerParams(needs_layout_passes=False),
        out_type=jax.ShapeDtypeStruct(((B // 2) * 2 * V,), jnp.float32),
        scratch_types=[
            pltpu.VMEM((L, chunkb), jnp.int32),
            pltpu.VMEM((L, chunkb), jnp.int32),
            pltpu.VMEM((L, chunkb), jnp.int32),
            pltpu.VMEM((L, chunkb), jnp.int32),
            pltpu.VMEM((subb * 2 * V,), jnp.float32),
            pltpu.VMEM((subb * 2 * V,), jnp.float32),
            pltpu.SemaphoreType.DMA,
            pltpu.SemaphoreType.DMA,
            pltpu.SemaphoreType.DMA,
            pltpu.SemaphoreType.DMA,
        ],
    )
    def hist(xt_hbm, out_hbm, xa0, xa1, xb0, xb1, cbuf0, cbuf1,
             in0, in1, out0, out1):
        wid = lax.axis_index("s") * NC + lax.axis_index("c")
        base2 = wid * half_rows_w
        zeros16 = jnp.zeros((LANES,), jnp.float32)
        ones16 = jnp.full((LANES,), 1.0, jnp.float32)
        row_step = lax.iota(jnp.int32, LANES) * (2 * V)
        xabufs = (xa0, xa1)
        xbbufs = (xb0, xb1)
        cbufs = (cbuf0, cbuf1)
        in_sems = (in0, in1)
        out_sems = (out0, out1)

        def start_in(c):
            col0 = base2 + c * chunkb
            sem = in_sems[c % 2]
            return [
                pltpu.async_copy(
                    xt_hbm.at[:, pl.ds(col0, chunkb)], xabufs[c % 2], sem),
                pltpu.async_copy(
                    xt_hbm.at[:, pl.ds(col0 + B // 2, chunkb)],
                    xbbufs[c % 2], sem),
            ]

        def start_out(s, sl):
            row0 = base2 + s * subb
            return pltpu.async_copy(
                cbufs[sl],
                out_hbm.at[pl.ds(row0 * 2 * V, subb * 2 * V)],
                out_sems[sl])

        pending_in = [start_in(0)]
        pending_out = [None, None]
        n_per_chunk = chunkb // subb
        n_subs = n_chunks * n_per_chunk
        zvecs = subb * 2 * V // LANES
        n_iters = L // 4
        zper = zvecs // n_iters
        ztail = zvecs - zper * n_iters

        @plsc.parallel_loop(0, zvecs, 1, unroll=8)
        def zero_body(r):
            cbuf0[pl.ds(r * LANES, LANES)] = zeros16
            cbuf1[pl.ds(r * LANES, LANES)] = zeros16

        for s in range(n_subs):
            c, sub = divmod(s, n_per_chunk)
            sl = s % 2
            if sub == 0:
                for h in pending_in[c]:
                    h.wait()
                if c + 1 < n_chunks:
                    pending_in.append(start_in(c + 1))
            zb = None
            if s >= 1 and s + 1 < n_subs:
                osl = sl ^ 1
                if pending_out[osl] is not None:
                    pending_out[osl].wait()
                    pending_out[osl] = None
                zb = cbufs[osl]
            xa = xabufs[c % 2]
            xb = xbbufs[c % 2]
            cb = cbufs[sl]

            def l_body(li, carry):
                work = []
                for u in range(4):
                    l = li * 4 + u
                    for p, xp in ((0, xa), (1, xb)):
                        for g in range(n_groups):
                            win = cb.at[pl.ds(
                                g * LANES * 2 * V + p * V,
                                (LANES - 1) * 2 * V + V)]
                            tok = xp[l, pl.ds(sub * subb + g * LANES,
                                              LANES)]
                            work.append((win, tok))
                idxs = [row_step + tok for _, tok in work]
                if zb is not None:
                    for z in range(zper):
                        zb[pl.ds((li * zper + z) * LANES, LANES)] = zeros16
                for (win, _), idx in zip(work, idxs):
                    plsc.addupdate_scatter(win, [idx], ones16)
                return carry

            lax.fori_loop(0, n_iters, l_body, 0)
            if zb is not None:
                for z in range(ztail):
                    zb[pl.ds((zper * n_iters + z) * LANES, LANES)] = zeros16
            pending_out[sl] = start_out(s, sl)
        for p in pending_out:
            if p is not None:
                p.wait()

    return hist


def _tc_head_body(inv_l, counts_ref, emb_ref, w_ref, b_ref, out_ref):
    m2 = lax.dot_general(emb_ref[...], w_ref[...],
                         (((1,), (1,)), ((), ())),
                         preferred_element_type=jnp.float32) * inv_l
    h = pl.program_id(0)
    m2s = jnp.concatenate([m2, m2], axis=0)
    rid = lax.broadcasted_iota(jnp.int32, (2 * V, V), 0)
    m2h = jnp.where((rid // V) == h, m2s, 0.0)
    out_ref[...] = lax.dot_general(m2h, counts_ref[...],
                                   (((0,), (1,)), ((), ())),
                                   preferred_element_type=jnp.float32) + b_ref[...]


def _tc_head(counts2, emb, W, bcol, L):
    Bh = counts2.shape[0]
    tb = min(Bh, 8192)
    nb = Bh // tb
    return pl.pallas_call(
        functools.partial(_tc_head_body, 1.0 / L),
        grid=(2, nb),
        in_specs=[
            pl.BlockSpec((tb, 2 * V), lambda h, j: (j, 0)),
            pl.BlockSpec((V, D), lambda h, j: (0, 0)),
            pl.BlockSpec((V, D), lambda h, j: (0, 0)),
            pl.BlockSpec((V, 1), lambda h, j: (0, 0)),
        ],
        out_specs=pl.BlockSpec((V, tb), lambda h, j: (0, h * nb + j)),
        out_shape=jax.ShapeDtypeStruct((V, 2 * Bh), jnp.float32),
    )(counts2, emb, W, bcol)


def kernel(x, emb, W, b):
    B, L = x.shape
    x = x.astype(jnp.int32)
    counts2 = _make_sc_histogram(B, L)(x.T).reshape(B // 2, 2 * V)
    out_t = _tc_head(counts2, emb, W, b.reshape(V, 1), L)
    return out_t.T
